# Initial kernel scaffold; baseline (speedup 1.0000x reference)
import jax
import jax.numpy as jnp
from jax.experimental import pallas as pl


def kernel(x, W0, b0, W1, b1, W2, b2):
    raise NotImplementedError("write your pallas kernel here")


# speedup vs baseline: 1.3152x; 1.3152x over previous
import functools

import jax
import jax.numpy as jnp
from jax.experimental import pallas as pl

_N_HIDDEN = 128
_N_MEMORY = 32
_BUDGET = 8
_BLOCK = 1024


def _fused_body(x_ref, w0x_ref, w0m_ref, b0_ref, w1_ref, b1_ref,
                w2h_ref, b2h_ref, w2a_ref, b2a_ref, out_ref):
    xb = x_ref[...]
    w0x = w0x_ref[...]
    w0m = w0m_ref[...]
    b0 = b0_ref[...]
    w1 = w1_ref[...]
    b1 = b1_ref[...]
    w2h = w2h_ref[...]
    b2h = b2h_ref[...]
    w2a = w2a_ref[...]
    b2a = b2a_ref[...]

    bb = xb.shape[0]
    col = jax.lax.broadcasted_iota(jnp.int32, (bb, _N_MEMORY), 1)
    mem = jnp.where(col == 0, jnp.float32(_BUDGET), jnp.float32(0.0))
    halted = jnp.zeros((bb, 1), dtype=jnp.bool_)
    fprobs = jnp.zeros((bb, 1), dtype=jnp.float32)

    for _ in range(_BUDGET):
        h = jnp.dot(xb, w0x, preferred_element_type=jnp.float32)
        h = h + jnp.dot(mem, w0m, preferred_element_type=jnp.float32)
        h = jax.nn.relu(h + b0)
        h = jax.nn.relu(jnp.dot(h, w1, preferred_element_type=jnp.float32) + b1)
        hid = jnp.dot(h, w2h, preferred_element_type=jnp.float32) + b2h
        aux = jnp.dot(h, w2a, preferred_element_type=jnp.float32) + b2a
        probs = aux[:, 0:1]
        halt_val = aux[:, 1:2]
        mem_new = aux[:, 2:2 + _N_MEMORY]
        active = jnp.logical_not(halted)
        xb = jnp.where(active, hid, xb)
        mem = jnp.where(active, mem_new, mem)
        halt_out = jax.nn.sigmoid(halt_val) > 0.5
        newly = jnp.logical_and(active, halt_out)
        fprobs = jnp.where(newly, probs, fprobs)
        halted = jnp.logical_or(halted, newly)

    out_ref[...] = jax.nn.sigmoid(fprobs)


@functools.partial(jax.jit, static_argnames=())
def kernel(x, W0, b0, W1, b1, W2, b2):
    batch = x.shape[0]
    w0x = W0[:_N_HIDDEN]
    w0m = W0[_N_HIDDEN:]
    w2h = W2[:, 2:2 + _N_HIDDEN]
    b2h = b2[2:2 + _N_HIDDEN].reshape(1, _N_HIDDEN)
    w2a = jnp.concatenate([W2[:, 0:2], W2[:, 2 + _N_HIDDEN:]], axis=1)
    b2a = jnp.concatenate([b2[0:2], b2[2 + _N_HIDDEN:]]).reshape(1, -1)
    b0r = b0.reshape(1, -1)
    b1r = b1.reshape(1, -1)

    grid = batch // _BLOCK
    rep = lambda i: (0, 0)
    probs = pl.pallas_call(
        _fused_body,
        grid=(grid,),
        in_specs=[
            pl.BlockSpec((_BLOCK, _N_HIDDEN), lambda i: (i, 0)),
            pl.BlockSpec(w0x.shape, rep),
            pl.BlockSpec(w0m.shape, rep),
            pl.BlockSpec(b0r.shape, rep),
            pl.BlockSpec(W1.shape, rep),
            pl.BlockSpec(b1r.shape, rep),
            pl.BlockSpec(w2h.shape, rep),
            pl.BlockSpec(b2h.shape, rep),
            pl.BlockSpec(w2a.shape, rep),
            pl.BlockSpec(b2a.shape, rep),
        ],
        out_specs=pl.BlockSpec((_BLOCK, 1), lambda i: (i, 0)),
        out_shape=jax.ShapeDtypeStruct((batch, 1), jnp.float32),
    )(x, w0x, w0m, b0r, W1, b1r, w2h, b2h, w2a, b2a)
    final_probs = probs.reshape(batch)
    n_iters = jnp.zeros((batch,), dtype=x.dtype)
    return (final_probs, n_iters)


# speedup vs baseline: 1.6769x; 1.2750x over previous
import jax
import jax.numpy as jnp
from jax import lax
from jax.experimental import pallas as pl

_N_HIDDEN = 128
_N_MEMORY = 32
_BUDGET = 8
_BLOCK = 1024


def _fused_body(x_ref, w0xT_ref, w0mT_ref, b0i_ref, b0_ref, w1T_ref, b1_ref,
                w2hT_ref, b2h_ref, w2aT_ref, b2a_ref, out_ref):
    xb = x_ref[...]
    w0xT = w0xT_ref[...]
    w0mT = w0mT_ref[...]
    b0i = b0i_ref[...]
    b0 = b0_ref[...]
    w1T = w1T_ref[...]
    b1 = b1_ref[...]
    w2hT = w2hT_ref[...]
    b2h = b2h_ref[...]
    w2aT = w2aT_ref[...]
    b2a = b2a_ref[...]

    bb = xb.shape[0]
    hidT = None
    memT = None
    halted = None
    fprobs = jnp.zeros((1, bb), dtype=jnp.float32)

    for it in range(_BUDGET):
        if it == 0:
            h = lax.dot_general(w0xT, xb, (((1,), (1,)), ((), ())),
                                preferred_element_type=jnp.float32) + b0i
        else:
            h = lax.dot_general(w0xT, hidT, (((1,), (0,)), ((), ())),
                                preferred_element_type=jnp.float32)
            h = h + lax.dot_general(w0mT, memT, (((1,), (0,)), ((), ())),
                                    preferred_element_type=jnp.float32)
            h = h + b0
        h = jax.nn.relu(h)
        h = jax.nn.relu(
            lax.dot_general(w1T, h, (((1,), (0,)), ((), ())),
                            preferred_element_type=jnp.float32) + b1)
        auxT = lax.dot_general(w2aT, h, (((1,), (0,)), ((), ())),
                               preferred_element_type=jnp.float32) + b2a
        probsT = auxT[0:1, :]
        haltvT = auxT[1:2, :]
        if it < _BUDGET - 1:
            hidT = lax.dot_general(w2hT, h, (((1,), (0,)), ((), ())),
                                   preferred_element_type=jnp.float32) + b2h
            memT = auxT[2:2 + _N_MEMORY, :]
        halt = haltvT > 0.0
        if it == 0:
            newly = halt
            halted = newly
        else:
            newly = jnp.logical_and(halt, jnp.logical_not(halted))
            halted = jnp.logical_or(halted, newly)
        fprobs = jnp.where(newly, probsT, fprobs)

    out_ref[...] = jax.nn.sigmoid(fprobs).reshape(1, 1, bb)


def kernel(x, W0, b0, W1, b1, W2, b2):
    batch = x.shape[0]
    w0xT = W0[:_N_HIDDEN].T
    w0mT = W0[_N_HIDDEN:].T
    b0c = b0.reshape(-1, 1)
    b0i = (b0 + jnp.float32(_BUDGET) * W0[_N_HIDDEN]).reshape(-1, 1)
    w1T = W1.T
    b1c = b1.reshape(-1, 1)
    w2hT = W2[:, 2:2 + _N_HIDDEN].T
    b2hc = b2[2:2 + _N_HIDDEN].reshape(-1, 1)
    w2aT = jnp.concatenate([W2[:, 0:2], W2[:, 2 + _N_HIDDEN:]], axis=1).T
    b2ac = jnp.concatenate([b2[0:2], b2[2 + _N_HIDDEN:]]).reshape(-1, 1)

    grid = batch // _BLOCK
    rep = lambda i: (0, 0)
    probs = pl.pallas_call(
        _fused_body,
        grid=(grid,),
        in_specs=[
            pl.BlockSpec((_BLOCK, _N_HIDDEN), lambda i: (i, 0)),
            pl.BlockSpec(w0xT.shape, rep),
            pl.BlockSpec(w0mT.shape, rep),
            pl.BlockSpec(b0i.shape, rep),
            pl.BlockSpec(b0c.shape, rep),
            pl.BlockSpec(w1T.shape, rep),
            pl.BlockSpec(b1c.shape, rep),
            pl.BlockSpec(w2hT.shape, rep),
            pl.BlockSpec(b2hc.shape, rep),
            pl.BlockSpec(w2aT.shape, rep),
            pl.BlockSpec(b2ac.shape, rep),
        ],
        out_specs=pl.BlockSpec((1, 1, _BLOCK), lambda i: (i, 0, 0)),
        out_shape=jax.ShapeDtypeStruct((grid, 1, _BLOCK), jnp.float32),
    )(x, w0xT, w0mT, b0i, b0c, w1T, b1c, w2hT, b2hc, w2aT, b2ac)
    final_probs = probs.reshape(batch)
    n_iters = jnp.zeros((batch,), dtype=x.dtype)
    return (final_probs, n_iters)
